# baseline (speedup 1.0000x reference)
import functools

import jax
import jax.numpy as jnp
from jax import lax
from jax.experimental import pallas as pl
from jax.experimental.pallas import tpu as pltpu
from jax.experimental.pallas import tpu_sc as plsc

_B = 16384
_D = 32
_NW = 32
_PER_W = _B // _NW
_NBLK = _PER_W // 128

_cache = {}


def _build():
    if "k" in _cache:
        return _cache["k"]
    mesh = plsc.VectorSubcoreMesh(core_axis_name="c", subcore_axis_name="s")

    @functools.partial(
        pl.kernel,
        mesh=mesh,
        out_type=jax.ShapeDtypeStruct((2 * _D, _B), jnp.float32),
        scratch_types=[
            pltpu.VMEM((8, 128), jnp.int32),
            pltpu.VMEM((2, 4, 2, _D, 128), jnp.float32),
            pltpu.VMEM((2, _D, 128), jnp.float32),
            pltpu.SemaphoreType.DMA,
            pltpu.SemaphoreType.DMA,
            pltpu.SemaphoreType.DMA,
        ],
        compiler_params=pltpu.CompilerParams(
            use_tc_tiling_on_sc=True, needs_layout_passes=False
        ),
    )
    def k(idx_hbm, tableT_hbm, out_hbm, idx_s, stage, colbuf, semA, semB, osem):
        wid = lax.axis_index("s") * 2 + lax.axis_index("c")
        pltpu.sync_copy(idx_hbm.at[wid], idx_s)
        sems = (semA, semB)
        iota16 = lax.iota(jnp.int32, 16)

        def fire(run, p):
            for kk in range(4):
                for e in range(2):
                    v = run[p * 8 + 4 * e + kk]
                    v128 = pl.multiple_of((v // 128) * 128, 128)
                    pltpu.async_copy(
                        tableT_hbm.at[:, pl.ds(v128, 128)],
                        stage.at[p, kk, e],
                        sems[p],
                    )

        def drain(p):
            for kk in range(4):
                for e in range(2):
                    pltpu.make_async_copy(
                        tableT_hbm.at[:, pl.ds(0, 128)],
                        stage.at[p, kk, e],
                        sems[p],
                    ).wait()

        def extract(run, p, jbase):
            for kk in range(4):
                jvec = jnp.full((16,), 0, jnp.int32) + (jbase + kk)
                for e in range(2):
                    v = run[p * 8 + 4 * e + kk]
                    c = v - (v // 128) * 128
                    cvec = jnp.full((16,), 0, jnp.int32) + c
                    for half in range(2):
                        rows = iota16 + (16 * half)
                        vals = plsc.load_gather(
                            stage.at[p, kk, e], [rows, cvec]
                        )
                        plsc.store_scatter(colbuf.at[e], [rows, jvec], vals)

        for blk in range(_NBLK):
            for sub in range(2):
                row_ref = idx_s.at[2 * blk + sub]

                run0 = row_ref[pl.ds(0, 16)]
                fire(run0, 0)

                def body(hh, _, sub=sub):
                    off = pl.multiple_of(hh * 16, 16)
                    run = row_ref[pl.ds(off, 16)]
                    fire(run, 1)
                    drain(0)
                    jbase = sub * 64 + hh * 8
                    extract(run, 0, jbase)

                    offn = pl.multiple_of(
                        jnp.minimum(hh + 1, 7) * 16, 16
                    )
                    runn = row_ref[pl.ds(offn, 16)]

                    @pl.when(hh < 7)
                    def _():
                        fire(runn, 0)

                    drain(1)
                    extract(run, 1, jbase + 4)
                    return ()

                lax.fori_loop(0, 8, body, ())

            jg = wid * _NBLK + blk
            writes = []
            for e in range(2):
                for i in range(4):
                    writes.append(
                        pltpu.async_copy(
                            colbuf.at[e, pl.ds(8 * i, 8)],
                            out_hbm.at[
                                pl.ds(e * _D + 8 * i, 8),
                                pl.ds(jg * 128, 128),
                            ],
                            osem,
                        )
                    )
            for wcp in writes:
                wcp.wait()

    _cache["k"] = k
    return k


def kernel(input, table):
    item_size = 2
    ev = input[:, 0].astype(jnp.int32).reshape(_NW, _NBLK, 2, 8, 2, 1, 4)
    cond = input[:, item_size].astype(jnp.int32).reshape(
        _NW, _NBLK, 2, 8, 2, 1, 4
    )
    idx = jnp.concatenate([ev, cond], axis=5).reshape(_NW, 8, 128)
    out = _build()(idx, table.T)
    return out.T
